# baseline (speedup 1.0000x reference)
import functools

import jax
import jax.numpy as jnp
from jax import lax
from jax.experimental import pallas as pl
from jax.experimental.pallas import tpu as pltpu
from jax.experimental.pallas import tpu_sc as plsc

STATE_SIZE = 100
NUM_CLASSES = 1000
OUT_FEATURES = 128
BATCH = 4096

NUM_CORES = 2
NUM_SUBCORES = 16
NUM_WORKERS = NUM_CORES * NUM_SUBCORES
BAGS_PER_WORKER = BATCH // NUM_WORKERS
ROWS_PER_WORKER = BAGS_PER_WORKER * STATE_SIZE
CHUNK = 128
NUM_CHUNKS = ROWS_PER_WORKER // CHUNK


@functools.partial(
    pl.kernel,
    out_type=jax.ShapeDtypeStruct((BATCH, OUT_FEATURES), jnp.float32),
    mesh=plsc.VectorSubcoreMesh(
        core_axis_name="c", subcore_axis_name="s",
        num_cores=NUM_CORES, num_subcores=NUM_SUBCORES,
    ),
    scratch_types=[
        pltpu.VMEM((NUM_CHUNKS, CHUNK), jnp.int32),
        pltpu.VMEM((NUM_CHUNKS, CHUNK), jnp.int32),
    ] + [pltpu.VMEM((CHUNK, OUT_FEATURES), jnp.float32)
         for _ in range(5)]
      + [pltpu.VMEM_SHARED((NUM_SUBCORES * BAGS_PER_WORKER, OUT_FEATURES),
                           jnp.float32)]
      + [pltpu.SemaphoreType.DMA] * 10,
)
def _embag(tok_hbm, bagid_hbm, w_hbm, binit_hbm, out_hbm,
           tok, bag, r0, r1, r2, r3, r4, acc,
           g0, g1, g2, g3, g4,
           s0, s1, s2, s3, s4):
    rows = [r0, r1, r2, r3, r4]
    gsem = [g0, g1, g2, g3, g4]
    ssem = [s0, s1, s2, s3, s4]
    RING = 5
    G = 3

    cid = lax.axis_index("c")
    sid = lax.axis_index("s")
    wid = cid * NUM_SUBCORES + sid
    base_bag = sid * BAGS_PER_WORKER

    def gather(j, b):
        pltpu.async_copy(w_hbm.at[tok.at[j]], rows[b], gsem[b])

    def gather_wait(j, b):
        pltpu.make_async_copy(w_hbm.at[tok.at[j]], rows[b], gsem[b]).wait()

    def scatter(j, b):
        pltpu.async_copy(rows[b], acc.at[bag.at[j]], ssem[b], add=True)

    def scatter_wait(j, b):
        pltpu.make_async_copy(rows[b], acc.at[bag.at[j]], ssem[b]).wait()

    pltpu.sync_copy(tok_hbm.at[wid], tok)
    pltpu.sync_copy(bagid_hbm.at[sid], bag)

    pltpu.sync_copy(binit_hbm, rows[0])
    pltpu.sync_copy(rows[0], acc.at[pl.ds(base_bag, BAGS_PER_WORKER)])

    for b in range(G):
        gather(b, b)

    for b in range(RING):
        b2 = (b + G) % RING
        if b + G >= RING:
            scatter_wait(b + G - RING, b2)
        gather(b + G, b2)
        gather_wait(b, b)
        scatter(b, b)

    def _lap(it, _):
        j0 = it * RING
        for b in range(RING):
            j = j0 + b
            b2 = (b + G) % RING
            scatter_wait(j + G - RING, b2)
            gather(j + G, b2)
            gather_wait(j, b)
            scatter(j, b)
        return 0

    lax.fori_loop(1, NUM_CHUNKS // RING - 1, _lap, 0)

    t0 = NUM_CHUNKS - RING
    for b in range(RING):
        j = t0 + b
        if j + G < NUM_CHUNKS:
            b2 = (b + G) % RING
            scatter_wait(j + G - RING, b2)
            gather(j + G, b2)
        gather_wait(j, b)
        scatter(j, b)
    for b in range(RING):
        scatter_wait(t0 + b, b)

    pltpu.sync_copy(acc.at[pl.ds(base_bag, BAGS_PER_WORKER)], rows[0])
    pltpu.sync_copy(rows[0], out_hbm.at[pl.ds(wid * BAGS_PER_WORKER,
                                              BAGS_PER_WORKER)])


def kernel(indices, weight, bias):
    offsets = jnp.arange(STATE_SIZE, dtype=indices.dtype) * NUM_CLASSES
    tokens = (indices + offsets[None, :]).astype(jnp.int32)
    tokens = tokens.reshape(NUM_WORKERS, NUM_CHUNKS, CHUNK)

    p = jnp.arange(ROWS_PER_WORKER, dtype=jnp.int32) // STATE_SIZE
    bagids = (p[None, :] +
              jnp.arange(NUM_SUBCORES, dtype=jnp.int32)[:, None] *
              BAGS_PER_WORKER)
    bagids = bagids.reshape(NUM_SUBCORES, NUM_CHUNKS, CHUNK)

    binit = jnp.broadcast_to(bias.astype(jnp.float32),
                             (BAGS_PER_WORKER, OUT_FEATURES))

    return _embag(tokens, bagids, weight, binit)


# speedup vs baseline: 1.5318x; 1.5318x over previous
import functools

import jax
import jax.numpy as jnp
from jax import lax
from jax.experimental import pallas as pl
from jax.experimental.pallas import tpu as pltpu
from jax.experimental.pallas import tpu_sc as plsc

STATE_SIZE = 100
NUM_CLASSES = 1000
OUT_FEATURES = 128
BATCH = 4096

NUM_CORES = 2
NUM_SUBCORES = 16
NUM_WORKERS = NUM_CORES * NUM_SUBCORES
BAGS_PER_WORKER = BATCH // NUM_WORKERS
LANE = 16
NVEC = OUT_FEATURES // LANE
RING = 4


@functools.partial(
    pl.kernel,
    out_type=jax.ShapeDtypeStruct((BATCH, OUT_FEATURES), jnp.float32),
    mesh=plsc.VectorSubcoreMesh(
        core_axis_name="c", subcore_axis_name="s",
        num_cores=NUM_CORES, num_subcores=NUM_SUBCORES,
    ),
    scratch_types=[
        pltpu.VMEM((BAGS_PER_WORKER, STATE_SIZE), jnp.int32),
        pltpu.VMEM((OUT_FEATURES,), jnp.float32),
        pltpu.VMEM((BAGS_PER_WORKER, OUT_FEATURES), jnp.float32),
    ] + [pltpu.VMEM((STATE_SIZE, OUT_FEATURES), jnp.float32)
         for _ in range(RING)]
      + [pltpu.SemaphoreType.DMA] * RING,
)
def _embag(tok_hbm, w_hbm, b_hbm, out_hbm,
           tok, bvec, outb, r0, r1, r2, r3, g0, g1, g2, g3):
    rows = [r0, r1, r2, r3]
    gsem = [g0, g1, g2, g3]

    cid = lax.axis_index("c")
    sid = lax.axis_index("s")
    wid = cid * NUM_SUBCORES + sid

    def gather(j, b):
        pltpu.async_copy(w_hbm.at[tok.at[j]], rows[b], gsem[b])

    def gather_wait(j, b):
        pltpu.make_async_copy(w_hbm.at[tok.at[j]], rows[b], gsem[b]).wait()

    pltpu.sync_copy(tok_hbm.at[wid], tok)
    pltpu.sync_copy(b_hbm, bvec)
    bias_v = [bvec[pl.ds(k * LANE, LANE)] for k in range(NVEC)]

    for b in range(RING):
        gather(b, b)

    def _reduce(j, b):
        buf = rows[b]

        def body(r, acc):
            return tuple(acc[k] + buf[r, pl.ds(k * LANE, LANE)]
                         for k in range(NVEC))

        acc = lax.fori_loop(0, STATE_SIZE, body, tuple(bias_v))
        for k in range(NVEC):
            outb[j, pl.ds(k * LANE, LANE)] = acc[k]

    def _lap(it, _):
        j0 = it * RING
        for b in range(RING):
            j = j0 + b
            gather_wait(j, b)
            _reduce(j, b)
            gather(j + RING, b)
        return 0

    lax.fori_loop(0, BAGS_PER_WORKER // RING - 1, _lap, 0)

    t0 = BAGS_PER_WORKER - RING
    for b in range(RING):
        gather_wait(t0 + b, b)
        _reduce(t0 + b, b)

    pltpu.sync_copy(outb, out_hbm.at[pl.ds(wid * BAGS_PER_WORKER,
                                           BAGS_PER_WORKER)])


def kernel(indices, weight, bias):
    offsets = jnp.arange(STATE_SIZE, dtype=indices.dtype) * NUM_CLASSES
    tokens = (indices + offsets[None, :]).astype(jnp.int32)
    tokens = tokens.reshape(NUM_WORKERS, BAGS_PER_WORKER, STATE_SIZE)
    return _embag(tokens, weight, bias.astype(jnp.float32))
